# baseline (speedup 1.0000x reference)
import functools
import math

import jax
import jax.numpy as jnp
from jax import lax
from jax.experimental import pallas as pl
from jax.experimental.pallas import tpu as pltpu
from jax.experimental.pallas import tpu_sc as plsc

N = 50000
D = 32
NUM_GRAPHS = 128
NUM_CLASSES = 5

ROW_TILE = 1792
NP = 50176
HEAD_STEPS = NP // ROW_TILE
JUNK = N

E = 800000
NW = 32
CHUNK = 64
SUP = 40
NSUP = 10
EPT = CHUNK * SUP * NSUP
EP = NW * EPT
NCHUNK = SUP * NSUP
NR = 50048
TILE_ROWS = NR // 16
QW_EDGE = 48
QW_PLAIN = 32

_INV_SQRT_D = 1.0 / math.sqrt(float(D))


def _proj_body(has_edge, xs_ref, xd_ref, Wq, bq, Wk, bk, Wv, bv, Ws, bs, We,
               q_ref, kv_ref, skip_ref, sv_ref):
    i = pl.program_id(0)
    xd = xd_ref[...]
    xs = xs_ref[...]
    q = (jnp.dot(xd, Wq[...], preferred_element_type=jnp.float32) + bq[...]) * _INV_SQRT_D
    k = jnp.dot(xs, Wk[...], preferred_element_type=jnp.float32) + bk[...]
    v = jnp.dot(xs, Wv[...], preferred_element_type=jnp.float32) + bv[...]
    kv_ref[...] = jnp.concatenate([k, v], axis=1)
    skip_ref[...] = jnp.dot(xd, Ws[...], preferred_element_type=jnp.float32) + bs[...]
    if has_edge:
        s = jnp.sum(We[...], axis=0, keepdims=True)
        t = jnp.dot(q, s.T, preferred_element_type=jnp.float32)
        pad = jnp.zeros((q.shape[0], QW_EDGE - D - 1), jnp.float32)
        q_ref[...] = jnp.concatenate([q, t, pad], axis=1)

        @pl.when(i == 0)
        def _sv():
            sv_ref[...] = s
    else:
        q_ref[...] = q


def _proj(xs, xd, Wq, bq, Wk, bk, Wv, bv, Ws, bs, We):
    has_edge = We is not None
    qw = QW_EDGE if has_edge else QW_PLAIN
    w_arrs = [Wq, bq, Wk, bk, Wv, bv, Ws, bs]
    if has_edge:
        w_arrs.append(We)
    else:
        w_arrs.append(jnp.zeros((1, 1), jnp.float32))
    row = lambda i: (i, 0)
    w_specs = [pl.BlockSpec(w.shape, lambda i, r=len(w.shape): (0,) * r) for w in w_arrs]
    out = pl.pallas_call(
        functools.partial(_proj_body, has_edge),
        grid=(HEAD_STEPS,),
        in_specs=[pl.BlockSpec((ROW_TILE, D), row), pl.BlockSpec((ROW_TILE, D), row)] + w_specs,
        out_specs=[
            pl.BlockSpec((ROW_TILE, qw), row),
            pl.BlockSpec((ROW_TILE, 2 * D), row),
            pl.BlockSpec((ROW_TILE, D), row),
            pl.BlockSpec((1, D), lambda i: (0, 0)),
        ],
        out_shape=[
            jax.ShapeDtypeStruct((NP, qw), jnp.float32),
            jax.ShapeDtypeStruct((NP, 2 * D), jnp.float32),
            jax.ShapeDtypeStruct((NP, D), jnp.float32),
            jax.ShapeDtypeStruct((1, D), jnp.float32),
        ],
    )(xs, xd, *w_arrs)
    return out


def _sc_edge_body(has_edge, q_hbm, kv_hbm, src_hbm, dst_hbm, w_hbm, sv_hbm,
                  z2_hbm, z1_hbm, num_hbm, den_hbm,
                  isup, dsup, wsup, qb0, qb1, kvb0, kvb1, msgb0, msgb1,
                  exb0, exb1, sb, num_s, den_s,
                  sq0, sq1, skv0, skv1, ssc0, ssc1):
    c = lax.axis_index("c")
    s = lax.axis_index("s")
    wid = s * 2 + c

    qb = (qb0, qb1)
    kvb = (kvb0, kvb1)
    msgb = (msgb0, msgb1)
    exb = (exb0, exb1)
    sq = (sq0, sq1)
    skv = (skv0, skv1)
    ssc = (ssc0, ssc1)

    pltpu.sync_copy(z2_hbm, num_s.at[pl.ds(s * TILE_ROWS, TILE_ROWS)])
    pltpu.sync_copy(z1_hbm, den_s.at[pl.ds(s * TILE_ROWS, TILE_ROWS)])

    @pl.when(s == 0)
    def _tail():
        pltpu.sync_copy(z2_hbm.at[pl.ds(0, NP - NR)], num_hbm.at[c].at[pl.ds(NR, NP - NR)])
        pltpu.sync_copy(z1_hbm.at[pl.ds(0, NP - NR)], den_hbm.at[c].at[pl.ds(NR, NP - NR)])

    if has_edge:
        pltpu.sync_copy(sv_hbm, sb)
        sv0 = sb[pl.ds(0, 16)]
        sv1 = sb[pl.ds(16, 16)]

    plsc.subcore_barrier()

    def _issue_gather(ci, b):
        pltpu.async_copy(q_hbm.at[dsup.at[ci]], qb[b], sq[b])
        pltpu.async_copy(kv_hbm.at[isup.at[ci]], kvb[b], skv[b])

    def _wait_gather(ci, b):
        pltpu.make_async_copy(q_hbm.at[dsup.at[ci]], qb[b], sq[b]).wait()
        pltpu.make_async_copy(kv_hbm.at[isup.at[ci]], kvb[b], skv[b]).wait()

    def _compute(ci, b):
        qr, kvr = qb[b], kvb[b]
        mr, er = msgb[b], exb[b]
        lane = jnp.arange(16, dtype=jnp.int32)

        def _group(g, _):
            j0 = g * 16
            av = jnp.zeros((16,), jnp.float32)
            if has_edge:
                tvv = jnp.zeros((16,), jnp.float32)
            for e in range(16):
                j = j0 + e
                q0 = qr[j, pl.ds(0, 16)]
                q1 = qr[j, pl.ds(16, 16)]
                k0 = kvr[j, pl.ds(0, 16)]
                k1 = kvr[j, pl.ds(16, 16)]
                dt = jnp.sum(q0 * k0 + q1 * k1)
                av = jnp.where(lane == e, dt, av)
                if has_edge:
                    q2 = qr[j, pl.ds(D, 16)]
                    tvv = jnp.where(lane == e, q2[0], tvv)
            if has_edge:
                wv = wsup[ci, pl.ds(j0, 16)]
                ex = jnp.exp(av + wv * tvv)
                ewv = ex * wv
            else:
                ex = jnp.exp(av)
            er[pl.ds(j0, 16)] = ex
            for e in range(16):
                j = j0 + e
                v0 = kvr[j, pl.ds(32, 16)]
                v1 = kvr[j, pl.ds(48, 16)]
                exe = ex[e]
                m0 = exe * v0
                m1 = exe * v1
                if has_edge:
                    ewe = ewv[e]
                    m0 = m0 + ewe * sv0
                    m1 = m1 + ewe * sv1
                mr[j, pl.ds(0, 16)] = m0
                mr[j, pl.ds(16, 16)] = m1
            return 0

        lax.fori_loop(0, CHUNK // 16, _group, 0)

    def _issue_scatter(ci, b):
        pltpu.async_copy(msgb[b], num_s.at[dsup.at[ci]], ssc[b], add=True)
        pltpu.async_copy(exb[b], den_s.at[dsup.at[ci]], ssc[b], add=True)

    def _wait_scatter(ci, b):
        pltpu.make_async_copy(msgb[b], num_s.at[dsup.at[ci]], ssc[b]).wait()
        pltpu.make_async_copy(exb[b], den_s.at[dsup.at[ci]], ssc[b]).wait()

    def _super(u, _):
        row = wid * (NSUP * SUP) + u * SUP
        pltpu.sync_copy(src_hbm.at[pl.ds(row, SUP)], isup)
        pltpu.sync_copy(dst_hbm.at[pl.ds(row, SUP)], dsup)
        if has_edge:
            pltpu.sync_copy(w_hbm.at[pl.ds(row, SUP)], wsup)
        _issue_gather(0, 0)
        _issue_gather(1, 1)

        def _pair(p_i, _):
            for b in range(2):
                ci = p_i * 2 + b
                _wait_gather(ci, b)

                @pl.when(ci >= 2)
                def _drain():
                    _wait_scatter(ci - 2, b)

                _compute(ci, b)
                _issue_scatter(ci, b)

                @pl.when(ci + 2 < SUP)
                def _next():
                    _issue_gather(ci + 2, b)
            return 0

        lax.fori_loop(0, SUP // 2, _pair, 0)
        _wait_scatter(SUP - 2, 0)
        _wait_scatter(SUP - 1, 1)
        return 0

    lax.fori_loop(0, NSUP, _super, 0)

    plsc.subcore_barrier()

    pltpu.sync_copy(num_s.at[pl.ds(s * TILE_ROWS, TILE_ROWS)],
                    num_hbm.at[c].at[pl.ds(s * TILE_ROWS, TILE_ROWS)])
    pltpu.sync_copy(den_s.at[pl.ds(s * TILE_ROWS, TILE_ROWS)],
                    den_hbm.at[c].at[pl.ds(s * TILE_ROWS, TILE_ROWS)])


def _make_sc_edge(has_edge):
    qw = QW_EDGE if has_edge else QW_PLAIN
    mesh = plsc.VectorSubcoreMesh(core_axis_name="c", subcore_axis_name="s",
                                  num_cores=2, num_subcores=16)
    scratch = [
        pltpu.VMEM((SUP, CHUNK), jnp.int32),
        pltpu.VMEM((SUP, CHUNK), jnp.int32),
        pltpu.VMEM((SUP, CHUNK), jnp.float32),
        pltpu.VMEM((CHUNK, qw), jnp.float32),
        pltpu.VMEM((CHUNK, qw), jnp.float32),
        pltpu.VMEM((CHUNK, 2 * D), jnp.float32),
        pltpu.VMEM((CHUNK, 2 * D), jnp.float32),
        pltpu.VMEM((CHUNK, D), jnp.float32),
        pltpu.VMEM((CHUNK, D), jnp.float32),
        pltpu.VMEM((CHUNK,), jnp.float32),
        pltpu.VMEM((CHUNK,), jnp.float32),
        pltpu.VMEM((D,), jnp.float32),
        pltpu.VMEM_SHARED((NR, D), jnp.float32),
        pltpu.VMEM_SHARED((NR,), jnp.float32),
        pltpu.SemaphoreType.DMA,
        pltpu.SemaphoreType.DMA,
        pltpu.SemaphoreType.DMA,
        pltpu.SemaphoreType.DMA,
        pltpu.SemaphoreType.DMA,
        pltpu.SemaphoreType.DMA,
    ]
    out_type = (
        jax.ShapeDtypeStruct((2, NP, D), jnp.float32),
        jax.ShapeDtypeStruct((2, NP), jnp.float32),
    )
    return pl.kernel(
        functools.partial(_sc_edge_body, has_edge),
        out_type=out_type,
        mesh=mesh,
        scratch_types=scratch,
        compiler_params=pltpu.CompilerParams(needs_layout_passes=False,
                                             use_tc_tiling_on_sc=False),
    )


def _fin_body(num_ref, den_ref, skip_ref, h_ref):
    num = num_ref[0] + num_ref[1]
    den = den_ref[0] + den_ref[1]
    h_ref[...] = jnp.tanh(num / (den + 1e-16) + skip_ref[...])


def _finalize(num, den3, skip):
    return pl.pallas_call(
        _fin_body,
        grid=(HEAD_STEPS,),
        in_specs=[
            pl.BlockSpec((2, ROW_TILE, D), lambda i: (0, i, 0)),
            pl.BlockSpec((2, ROW_TILE, 1), lambda i: (0, i, 0)),
            pl.BlockSpec((ROW_TILE, D), lambda i: (i, 0)),
        ],
        out_specs=pl.BlockSpec((ROW_TILE, D), lambda i: (i, 0)),
        out_shape=jax.ShapeDtypeStruct((NP, D), jnp.float32),
    )(num, den3, skip)


def _head_body(h_refs, batch_ref, w_refs, out_ref, acc_ref, cnt_ref):
    i = pl.program_id(0)
    (W1, b1, W2, b2, W3, b3, W4, b4, W5, b5, W6, b6) = w_refs

    @pl.when(i == 0)
    def _init():
        acc_ref[...] = jnp.zeros_like(acc_ref)
        cnt_ref[...] = jnp.zeros_like(cnt_ref)

    cs = jnp.concatenate([r[...] for r in h_refs], axis=1)
    a1 = jnp.dot(cs, W1[...], preferred_element_type=jnp.float32) + b1[...]
    b = batch_ref[...][:, 0]
    gid = jax.lax.broadcasted_iota(jnp.int32, (NUM_GRAPHS, ROW_TILE), 0)
    oh = (b[None, :] == gid).astype(jnp.float32)
    acc_ref[...] += jnp.dot(oh, a1, preferred_element_type=jnp.float32)
    cnt_ref[...] += jnp.sum(oh, axis=1)[None, :]

    @pl.when(i == HEAD_STEPS - 1)
    def _final():
        cnt = cnt_ref[...][0]
        pooled = acc_ref[...] / jnp.maximum(cnt, 1.0)[:, None]
        h2 = jax.nn.relu(jnp.dot(pooled, W2[...], preferred_element_type=jnp.float32) + b2[...])
        h3 = jnp.dot(h2, W3[...], preferred_element_type=jnp.float32) + b3[...]
        h4 = jnp.dot(h3, W4[...], preferred_element_type=jnp.float32) + b4[...]
        h5 = jnp.dot(h4, W5[...], preferred_element_type=jnp.float32) + b5[...]
        h6 = jnp.dot(h5, W6[...], preferred_element_type=jnp.float32) + b6[...]
        m = jnp.max(h6, axis=-1, keepdims=True)
        lse = jnp.log(jnp.sum(jnp.exp(h6 - m), axis=-1, keepdims=True)) + m
        out_ref[...] = h6 - lse


def _head(hs, batch_pad, W1, b1, W2, b2, W3, b3, W4, b4, W5, b5, W6, b6):
    n_h = len(hs)

    def body(*refs):
        h_refs = refs[:n_h]
        batch_ref = refs[n_h]
        w_refs = refs[n_h + 1:n_h + 13]
        out_ref = refs[n_h + 13]
        acc_ref, cnt_ref = refs[n_h + 14:]
        _head_body(h_refs, batch_ref, w_refs, out_ref, acc_ref, cnt_ref)

    h_specs = [pl.BlockSpec((ROW_TILE, D), lambda i: (i, 0)) for _ in range(n_h)]
    b_spec = pl.BlockSpec((ROW_TILE, 1), lambda i: (i, 0))
    w_arrs = [W1, b1, W2, b2, W3, b3, W4, b4, W5, b5, W6, b6]
    w_specs = [pl.BlockSpec(w.shape, lambda i, r=len(w.shape): (0,) * r) for w in w_arrs]
    out = pl.pallas_call(
        body,
        grid=(HEAD_STEPS,),
        in_specs=h_specs + [b_spec] + w_specs,
        out_specs=pl.BlockSpec((NUM_GRAPHS, NUM_CLASSES), lambda i: (0, 0)),
        out_shape=jax.ShapeDtypeStruct((NUM_GRAPHS, NUM_CLASSES), jnp.float32),
        scratch_shapes=[
            pltpu.VMEM((NUM_GRAPHS, 288), jnp.float32),
            pltpu.VMEM((1, NUM_GRAPHS), jnp.float32),
        ],
    )(*hs, batch_pad, *w_arrs)
    return out


def _pad_edges(ei, attr=None):
    src = jnp.pad(ei[0], (0, EP - E), constant_values=JUNK).reshape(EP // CHUNK, CHUNK)
    dst = jnp.pad(ei[1], (0, EP - E), constant_values=JUNK).reshape(EP // CHUNK, CHUNK)
    w = None if attr is None else jnp.pad(attr, (0, EP - E)).reshape(EP // CHUNK, CHUNK)
    return src, dst, w


def kernel(x, edge_index, edge_attr, uv_target_index, uv_target_emb, target_uv_index, batch,
           Wq, bq, Wk, bk, Wv, bv, Ws, bs, We, W1, b1, W2, b2, W3, b3, W4, b4, W5, b5, W6, b6):
    xpad = jnp.pad(x, ((0, NP - N), (0, 0)))
    f0 = xpad[:, :D]
    f1 = xpad[:, D:]
    uvpad = jnp.pad(uv_target_emb, ((0, NP - N), (0, 0)))

    src_e, dst_e, w_e = _pad_edges(edge_index, edge_attr)
    src_u, dst_u, _ = _pad_edges(uv_target_index)
    src_t, dst_t, _ = _pad_edges(target_uv_index)

    sc_edge = _make_sc_edge(True)
    sc_plain = _make_sc_edge(False)
    dummy_sv = jnp.zeros((D,), jnp.float32)
    z2 = jnp.zeros((TILE_ROWS, D), jnp.float32)
    z1 = jnp.zeros((TILE_ROWS,), jnp.float32)

    cur = None
    hs = []
    e_idx = 0
    for i in range(9):
        b_q = bq[i].reshape(1, D)
        b_k = bk[i].reshape(1, D)
        b_v = bv[i].reshape(1, D)
        b_s = bs[i].reshape(1, D)
        if i % 3 == 0:
            xs, xd = (f0, f1) if i == 0 else (cur, cur)
            qt, kvt, skip, sv = _proj(xs, xd, Wq[i], b_q, Wk[i], b_k, Wv[i], b_v,
                                      Ws[i], b_s, We[e_idx])
            num, den = sc_edge(qt, kvt, src_e, dst_e, w_e, sv[0], z2, z1)
            e_idx += 1
        elif i % 3 == 1:
            xs, xd = cur, uvpad
            qt, kvt, skip, _ = _proj(xs, xd, Wq[i], b_q, Wk[i], b_k, Wv[i], b_v,
                                     Ws[i], b_s, None)
            num, den = sc_plain(qt, kvt, src_u, dst_u, src_u, dummy_sv, z2, z1)
        else:
            xs, xd = uvpad, cur
            qt, kvt, skip, _ = _proj(xs, xd, Wq[i], b_q, Wk[i], b_k, Wv[i], b_v,
                                     Ws[i], b_s, None)
            num, den = sc_plain(qt, kvt, src_t, dst_t, src_t, dummy_sv, z2, z1)
        den3 = den.reshape(2, NP, 1)
        h = _finalize(num, den3, skip)
        cur = h
        hs.append(h)

    batch_pad = jnp.pad(batch, (0, NP - N), constant_values=NUM_GRAPHS).reshape(NP, 1)
    return _head(hs, batch_pad, W1, b1, W2, b2, W3, b3, W4, b4, W5, b5, W6, b6)
